# Initial kernel scaffold; baseline (speedup 1.0000x reference)
import jax
import jax.numpy as jnp
from jax.experimental import pallas as pl


def kernel(x, edge_index, W, b):
    raise NotImplementedError("write your pallas kernel here")


# speedup vs baseline: 5.2367x; 5.2367x over previous
import functools
import math

import jax
import jax.numpy as jnp
from jax import lax
from jax.experimental import pallas as pl
from jax.experimental.pallas import tpu as pltpu
from jax.experimental.pallas import tpu_sc as plsc

N = 10000
N2 = 10240
E = 160000
T = 12
CIN = 256
H = 512
F = T * CIN
NCHUNK = 24
FC = 128
NTILES = 16
NCORES = 2
B = 128
NB = 80
EPT = NB * B
DEG_NB = 40
DEG_EPT = DEG_NB * B

_MESH = dict(core_axis_name="c", subcore_axis_name="s",
             num_cores=NCORES, num_subcores=NTILES)


def _deg_body(src_hbm, deg_hbm, deg_sh, idx_v, ones_v, zb_v):
    c = lax.axis_index("c")
    s = lax.axis_index("s")
    w = c * NTILES + s
    for i in range(B // 16):
        ones_v[pl.ds(i * 16, 16)] = jnp.full((16,), 1.0, jnp.float32)
    for i in range(640 // 16):
        zb_v[pl.ds(i * 16, 16)] = jnp.zeros((16,), jnp.float32)
    pltpu.sync_copy(zb_v, deg_sh.at[pl.ds(s * 640, 640)])
    plsc.subcore_barrier()
    pltpu.sync_copy(src_hbm.at[pl.ds(w * DEG_NB, DEG_NB)], idx_v)

    def body(j, carry):
        pltpu.sync_copy(ones_v, deg_sh.at[idx_v.at[j]], add=True)
        return carry

    lax.fori_loop(0, DEG_NB, body, 0)
    plsc.subcore_barrier()
    pltpu.sync_copy(deg_sh.at[pl.ds(s * 640, 640)],
                    deg_hbm.at[c, pl.ds(s * 640, 640)])


@functools.lru_cache(maxsize=None)
def _get_deg():
    return pl.kernel(
        _deg_body,
        out_type=jax.ShapeDtypeStruct((NCORES, N2), jnp.float32),
        mesh=plsc.VectorSubcoreMesh(**_MESH),
        scratch_types=[
            pltpu.VMEM_SHARED((N2,), jnp.float32),
            pltpu.VMEM((DEG_NB, B), jnp.int32),
            pltpu.VMEM((B,), jnp.float32),
            pltpu.VMEM((640,), jnp.float32),
        ],
    )


def _spmv_body(write_v, *refs):
    if write_v:
        (xc, src, dst, dinv, u, v,
         A_sh, src_adj, dst_v, rb, zb, dinv_v, sem) = refs
    else:
        (xc, src, dst, u,
         A_sh, src_adj, dst_v, rb, zb, dinv_v, sem) = refs
        dinv = v = None
    c = lax.axis_index("c")
    s = lax.axis_index("s")
    row0 = s * 640
    pltpu.sync_copy(dst.at[pl.ds(s * NB, NB)], dst_v)
    if write_v:
        pltpu.sync_copy(dinv.at[pl.ds(row0, 640)], dinv_v)

    for i in range(16):
        for q in range(FC // 16):
            zb[i, pl.ds(q * 16, 16)] = jnp.zeros((16,), jnp.float32)

    def chunk_body(ci, carry):
        ch = c * (NCHUNK // NCORES) + ci

        def zero_body(k, carry2):
            pltpu.sync_copy(zb, A_sh.at[pl.ds(row0 + k * 16, 16)])
            return carry2

        lax.fori_loop(0, 640 // 16, zero_body, 0)
        plsc.subcore_barrier()
        off = ch * N2
        pltpu.sync_copy(src.at[pl.ds(s * EPT, EPT)], src_adj)

        def adj_body(i, carry2):
            src_adj[pl.ds(i * 16, 16)] = src_adj[pl.ds(i * 16, 16)] + off
            return carry2

        lax.fori_loop(0, EPT // 16, adj_body, 0)

        def batch_body(j, carry2):
            pltpu.async_copy(xc.at[src_adj.at[pl.ds(j * B, B)]], rb, sem).wait()
            pltpu.sync_copy(rb, A_sh.at[dst_v.at[j]], add=True)
            return carry2

        lax.fori_loop(0, NB, batch_body, 0)
        plsc.subcore_barrier()

        def flush_body(k, carry2):
            r0 = row0 + k * B
            pltpu.sync_copy(A_sh.at[pl.ds(r0, B)], rb)
            pltpu.sync_copy(rb, u.at[pl.ds(r0, B), pl.ds(ch * FC, FC)])
            if write_v:
                def scale_body(g, carry3):
                    dvec = dinv_v[pl.ds(k * B + g * 16, 16)]
                    d2vec = dvec * dvec
                    for i in range(16):
                        row = g * 16 + i
                        d2 = d2vec[i]
                        for q in range(FC // 16):
                            rb[row, pl.ds(q * 16, 16)] = (
                                rb[row, pl.ds(q * 16, 16)] * d2)
                    return carry3

                lax.fori_loop(0, B // 16, scale_body, 0)
                pltpu.sync_copy(rb, v.at[pl.ds(off + r0, B)])
            return carry2

        lax.fori_loop(0, 5, flush_body, 0)
        plsc.subcore_barrier()
        return carry

    lax.fori_loop(0, NCHUNK // NCORES, chunk_body, 0)


@functools.lru_cache(maxsize=None)
def _make_spmv(write_v):
    outs = [jax.ShapeDtypeStruct((N2, F), jnp.float32)]
    if write_v:
        outs.append(jax.ShapeDtypeStruct((NCHUNK * N2, FC), jnp.float32))
    return pl.kernel(
        functools.partial(_spmv_body, write_v),
        out_type=tuple(outs) if write_v else outs[0],
        mesh=plsc.VectorSubcoreMesh(**_MESH),
        scratch_types=[
            pltpu.VMEM_SHARED((N2, FC), jnp.float32),
            pltpu.VMEM((EPT,), jnp.int32),
            pltpu.VMEM((NB, B), jnp.int32),
            pltpu.VMEM((B, FC), jnp.float32),
            pltpu.VMEM((16, FC), jnp.float32),
            pltpu.VMEM((640,), jnp.float32),
            pltpu.SemaphoreType.DMA,
        ],
    )


def _prep_body(degT_ref, x_ref, dinv_ref, xs_ref):
    d = degT_ref[:, 0:1] + degT_ref[:, 1:2]
    dinv = jnp.where(d > 0, lax.rsqrt(jnp.maximum(d, 1e-12)), 0.0)
    dinv_ref[...] = dinv
    xs_ref[...] = x_ref[...] * dinv


_BM = 128


def _prep(degT, x_pad):
    return pl.pallas_call(
        _prep_body,
        grid=(N2 // _BM,),
        in_specs=[
            pl.BlockSpec((_BM, 2), lambda i: (i, 0)),
            pl.BlockSpec((_BM, F), lambda i: (i, 0)),
        ],
        out_specs=[
            pl.BlockSpec((_BM, 1), lambda i: (i, 0)),
            pl.BlockSpec((_BM, F), lambda i: (i, 0)),
        ],
        out_shape=[
            jax.ShapeDtypeStruct((N2, 1), jnp.float32),
            jax.ShapeDtypeStruct((N2, F), jnp.float32),
        ],
    )(degT, x_pad)


def _out_body(x_ref, u1_ref, u2_ref, dinv_ref, w0_ref, w1_ref, w2_ref, b_ref,
              re_ref, im_ref):
    dv = dinv_ref[...]
    wcat = jnp.concatenate(
        [w0_ref[...] - w2_ref[...], -w1_ref[...], 2.0 * w2_ref[...]], axis=0)
    bias = b_ref[...]
    xb = x_ref[...]
    u1 = u1_ref[...] * dv
    u2 = u2_ref[...] * dv
    ys = []
    for t in range(T):
        sl = slice(t * CIN, (t + 1) * CIN)
        zt = jnp.concatenate([xb[:, sl], u1[:, sl], u2[:, sl]], axis=1)
        ys.append(jnp.dot(zt, wcat, preferred_element_type=jnp.float32) + bias)
    for f in range(T):
        re = None
        im = None
        for t in range(T):
            cv = math.cos(2.0 * math.pi * f * t / T)
            sv = -math.sin(2.0 * math.pi * f * t / T)
            if abs(cv) > 1e-9:
                term = ys[t] if abs(cv - 1.0) < 1e-9 else ys[t] * cv
                re = term if re is None else re + term
            if abs(sv) > 1e-9:
                term = ys[t] * sv
                im = term if im is None else im + term
        if im is None:
            im = jnp.zeros_like(ys[0])
        re_ref[:, f, :] = re
        im_ref[:, f, :] = im


def _outk(x_pad, u1, u2, dinv_col, w0, w1, w2, b2):
    wspec = pl.BlockSpec((CIN, H), lambda i: (0, 0))
    return pl.pallas_call(
        _out_body,
        grid=(N2 // _BM,),
        in_specs=[
            pl.BlockSpec((_BM, F), lambda i: (i, 0)),
            pl.BlockSpec((_BM, F), lambda i: (i, 0)),
            pl.BlockSpec((_BM, F), lambda i: (i, 0)),
            pl.BlockSpec((_BM, 1), lambda i: (i, 0)),
            wspec, wspec, wspec,
            pl.BlockSpec((1, H), lambda i: (0, 0)),
        ],
        out_specs=[
            pl.BlockSpec((_BM, T, H), lambda i: (i, 0, 0)),
            pl.BlockSpec((_BM, T, H), lambda i: (i, 0, 0)),
        ],
        out_shape=[
            jax.ShapeDtypeStruct((N2, T, H), jnp.float32),
            jax.ShapeDtypeStruct((N2, T, H), jnp.float32),
        ],
    )(x_pad, u1, u2, dinv_col, w0, w1, w2, b2)


def kernel(x, edge_index, W, b):
    x = x.astype(jnp.float32)
    src = edge_index[0].astype(jnp.int32)
    dst = edge_index[1].astype(jnp.int32)

    ept0 = E // NTILES
    pad16 = (N + (jnp.arange(EPT - ept0, dtype=jnp.int32) % (N2 - N)))
    pad16 = jnp.broadcast_to(pad16, (NTILES, EPT - ept0))
    src_p = jnp.concatenate([src.reshape(NTILES, ept0), pad16], axis=1).reshape(-1)
    dst_p = jnp.concatenate([dst.reshape(NTILES, ept0), pad16], axis=1)
    dst_p = dst_p.reshape(NTILES * NB, B)

    ept0d = E // 32
    pad32 = (N + (jnp.arange(DEG_EPT - ept0d, dtype=jnp.int32) % (N2 - N)))
    pad32 = jnp.broadcast_to(pad32, (32, DEG_EPT - ept0d))
    src_d = jnp.concatenate([src.reshape(32, ept0d), pad32], axis=1)
    src_d = src_d.reshape(32 * DEG_NB, B)

    deg2 = _get_deg()(src_d)
    x_pad = jnp.pad(x.reshape(N, F), ((0, N2 - N), (0, 0)))
    dinv_col, xs = _prep(deg2.T, x_pad)
    xc = xs.reshape(N2, NCHUNK, FC).transpose(1, 0, 2).reshape(NCHUNK * N2, FC)
    u1, v1 = _make_spmv(True)(xc, src_p, dst_p, dinv_col.reshape(N2))
    u2 = _make_spmv(False)(v1, src_p, dst_p)
    re, im = _outk(x_pad, u1, u2, dinv_col, W[0], W[1], W[2], b.reshape(1, H))
    return lax.complex(re[:N], im[:N])
